# baseline (speedup 1.0000x reference)
import functools

import jax
import jax.numpy as jnp
from jax import lax
from jax.experimental import pallas as pl
from jax.experimental.pallas import tpu as pltpu
from jax.experimental.pallas import tpu_sc as plsc

N = 10000
E = 320000
D_IN = 128
D_HID = 64
N_CLASSES = 41

NC = 2
NS = 16
LANES = 16
NW = NC * NS

CHUNK = 128
NCH = E // CHUNK
CPT = -(-NCH // NW)
NPAD = 10112
RPT = NPAD // NS
D2 = 64
NRING = 8

_mesh = plsc.VectorSubcoreMesh(core_axis_name="c", subcore_axis_name="s")
_sc_params = pltpu.CompilerParams(use_tc_tiling_on_sc=False)


def _chunk_range(wid):
    c0 = (wid * NCH) // NW
    c1 = ((wid + 1) * NCH) // NW
    return c0, c1 - c0


DLANES = 32

@functools.partial(
    pl.kernel,
    out_type=jax.ShapeDtypeStruct((NC, NPAD, DLANES), jnp.bfloat16),
    mesh=_mesh,
    scratch_types=[
        pltpu.VMEM((CPT, CHUNK), jnp.int32),
        pltpu.VMEM((CHUNK, DLANES), jnp.bfloat16),
        pltpu.VMEM_SHARED((NPAD, DLANES), jnp.bfloat16),
        pltpu.SemaphoreType.DMA,
    ],
    compiler_params=_sc_params,
)
def _deg_kernel(e3, ones_h, zeros_h, out, dst_v, ones_v, acc, ssem):
    c = lax.axis_index("c")
    s = lax.axis_index("s")
    wid = c * NS + s
    c0, tw = _chunk_range(wid)
    pltpu.sync_copy(e3.at[1, pl.ds(c0, CPT)], dst_v)
    pltpu.sync_copy(ones_h, ones_v)
    base = s * RPT
    pltpu.sync_copy(zeros_h.at[pl.ds(base, RPT)], acc.at[pl.ds(base, RPT)])
    plsc.subcore_barrier()

    def fire(j, carry):
        pltpu.async_copy(ones_v, acc.at[dst_v.at[j]], ssem, add=True)
        return carry

    lax.fori_loop(0, tw, fire, 0)

    def drain(j, carry):
        pltpu.make_async_copy(ones_v, acc.at[dst_v.at[0]], ssem).wait()
        return carry

    lax.fori_loop(0, tw, drain, 0)
    plsc.subcore_barrier()
    pltpu.sync_copy(acc.at[pl.ds(base, RPT)], out.at[c, pl.ds(base, RPT)])


def _make_agg(D):
    @functools.partial(
        pl.kernel,
        out_type=jax.ShapeDtypeStruct((NC, NPAD, D), jnp.bfloat16),
        mesh=_mesh,
        scratch_types=[
            pltpu.VMEM((CPT, CHUNK), jnp.int32),
            pltpu.VMEM((CPT, CHUNK), jnp.int32),
            [pltpu.VMEM((CHUNK, D), jnp.bfloat16)] * NRING,
            pltpu.VMEM_SHARED((NPAD, D), jnp.bfloat16),
            [pltpu.SemaphoreType.DMA] * NRING,
            [pltpu.SemaphoreType.DMA] * NRING,
        ],
        compiler_params=_sc_params,
    )
    def agg(e3, h_hbm, zeros_h, out, src_v, dst_v, bufs, acc, gs, ss):
        c = lax.axis_index("c")
        s = lax.axis_index("s")
        wid = c * NS + s
        c0, tw = _chunk_range(wid)
        pltpu.sync_copy(e3.at[0, pl.ds(c0, CPT)], src_v)
        pltpu.sync_copy(e3.at[1, pl.ds(c0, CPT)], dst_v)
        base = s * RPT
        pltpu.sync_copy(zeros_h.at[pl.ds(base, RPT)], acc.at[pl.ds(base, RPT)])
        plsc.subcore_barrier()

        for b in range(NRING):
            pltpu.async_copy(h_hbm.at[src_v.at[b]], bufs[b], gs[b])

        def body(t, carry):
            for b in range(NRING):
                j = NRING * t + b

                @pl.when(j < tw)
                def _():
                    pltpu.make_async_copy(h_hbm.at[src_v.at[j]], bufs[b], gs[b]).wait()
                    pltpu.async_copy(bufs[b], acc.at[dst_v.at[j]], ss[b], add=True)

            for b in range(NRING):
                jn = NRING * t + b + NRING

                @pl.when(jn < tw)
                def _():
                    pltpu.make_async_copy(bufs[b], acc.at[dst_v.at[0]], ss[b]).wait()
                    pltpu.async_copy(h_hbm.at[src_v.at[jn]], bufs[b], gs[b])

            return carry

        lax.fori_loop(0, -(-CPT // NRING), body, 0)
        for b in range(NRING):
            pltpu.make_async_copy(bufs[b], acc.at[dst_v.at[0]], ss[b]).wait()
        plsc.subcore_barrier()
        pltpu.sync_copy(acc.at[pl.ds(base, RPT)], out.at[c, pl.ds(base, RPT)])

    return agg


_agg = _make_agg(D_HID)


BR = 1264
GRID = NPAD // BR


def _dinv_of(degp_ref):
    deg = degp_ref[0, :, 0:1].astype(jnp.float32) + degp_ref[1, :, 0:1].astype(jnp.float32) + 1.0
    return lax.rsqrt(deg)


def _mm1_body(degp_ref, x_ref, w1_ref, h1p_ref, dinv_ref):
    i = pl.program_id(0)
    dinv = _dinv_of(degp_ref)
    h = jnp.dot(x_ref[...], w1_ref[...], preferred_element_type=jnp.float32)
    rows = i * BR + lax.broadcasted_iota(jnp.int32, (BR, 1), 0)
    h1p_ref[...] = jnp.where(rows < N, h * dinv, 0.0).astype(jnp.bfloat16)
    dinv_ref[...] = jnp.broadcast_to(dinv, (BR, 8))


def _mm2_body(dinv_ref, y_ref, h1p_ref, b1_ref, w2_ref, h2p_ref):
    dinv = dinv_ref[:, 0:1]
    ysum = (y_ref[0] + y_ref[1]).astype(jnp.float32) + h1p_ref[...].astype(jnp.float32)
    agg = dinv * ysum + b1_ref[...]
    o = jnp.maximum(agg, 0.0)
    h2 = jnp.dot(o, w2_ref[...], preferred_element_type=jnp.float32) * dinv
    h2p_ref[...] = jnp.pad(h2, ((0, 0), (0, D2 - N_CLASSES))).astype(jnp.bfloat16)


def _out_body(dinv_ref, y_ref, h2p_ref, b2_ref, out_ref):
    dinv = dinv_ref[:, 0:1]
    ysum = (y_ref[0] + y_ref[1]).astype(jnp.float32) + h2p_ref[...].astype(jnp.float32)
    h = (dinv * ysum)[:, :N_CLASSES] + b2_ref[...]
    m = jnp.max(h, axis=1, keepdims=True)
    lse = jnp.log(jnp.sum(jnp.exp(h - m), axis=1, keepdims=True)) + m
    out_ref[...] = h - lse


def _deg_spec():
    return pl.BlockSpec((NC, BR, DLANES), lambda i: (0, i, 0))


_mm1 = pl.pallas_call(
    _mm1_body,
    grid=(GRID,),
    in_specs=[
        _deg_spec(),
        pl.BlockSpec((BR, D_IN), lambda i: (i, 0)),
        pl.BlockSpec((D_IN, D_HID), lambda i: (0, 0)),
    ],
    out_specs=[
        pl.BlockSpec((BR, D_HID), lambda i: (i, 0)),
        pl.BlockSpec((BR, 8), lambda i: (i, 0)),
    ],
    out_shape=[
        jax.ShapeDtypeStruct((NPAD, D_HID), jnp.bfloat16),
        jax.ShapeDtypeStruct((NPAD, 8), jnp.float32),
    ],
)

_mm2 = pl.pallas_call(
    _mm2_body,
    grid=(GRID,),
    in_specs=[
        pl.BlockSpec((BR, 8), lambda i: (i, 0)),
        pl.BlockSpec((NC, BR, D_HID), lambda i: (0, i, 0)),
        pl.BlockSpec((BR, D_HID), lambda i: (i, 0)),
        pl.BlockSpec((1, D_HID), lambda i: (0, 0)),
        pl.BlockSpec((D_HID, N_CLASSES), lambda i: (0, 0)),
    ],
    out_specs=pl.BlockSpec((BR, D2), lambda i: (i, 0)),
    out_shape=jax.ShapeDtypeStruct((NPAD, D2), jnp.bfloat16),
)

_outk = pl.pallas_call(
    _out_body,
    grid=(GRID,),
    in_specs=[
        pl.BlockSpec((BR, 8), lambda i: (i, 0)),
        pl.BlockSpec((NC, BR, D2), lambda i: (0, i, 0)),
        pl.BlockSpec((BR, D2), lambda i: (i, 0)),
        pl.BlockSpec((1, N_CLASSES), lambda i: (0, 0)),
    ],
    out_specs=pl.BlockSpec((BR, N_CLASSES), lambda i: (i, 0)),
    out_shape=jax.ShapeDtypeStruct((N, N_CLASSES), jnp.float32),
)


@jax.jit
def kernel(x, edge_index, W1, b1, W2, b2):
    e3 = edge_index.astype(jnp.int32).reshape(2, NCH, CHUNK)

    ones_h = jnp.ones((CHUNK, DLANES), jnp.bfloat16)
    degp = _deg_kernel(e3, ones_h, jnp.zeros((NPAD, DLANES), jnp.bfloat16))

    h1p, dinv = _mm1(degp, x, W1)
    zeros_bf = jnp.zeros((NPAD, D_HID), jnp.bfloat16)
    y1 = _agg(e3, h1p, zeros_bf)

    h2p = _mm2(dinv, y1, h1p, b1.reshape(1, D_HID), W2)
    y2 = _agg(e3, h2p, zeros_bf)

    return _outk(dinv, y2, h2p, b2.reshape(1, N_CLASSES))


# speedup vs baseline: 1.0486x; 1.0486x over previous
import functools

import jax
import jax.numpy as jnp
from jax import lax
from jax.experimental import pallas as pl
from jax.experimental.pallas import tpu as pltpu
from jax.experimental.pallas import tpu_sc as plsc

N = 10000
E = 320000
D_IN = 128
D_HID = 64
N_CLASSES = 41

NC = 2
NS = 16
LANES = 16
NW = NC * NS

CHUNK = 128
NCH = E // CHUNK
CPT = -(-NCH // NW)
NPAD = 10112
RPT = NPAD // NS
D2 = 64
NRING = 8

_mesh = plsc.VectorSubcoreMesh(core_axis_name="c", subcore_axis_name="s")
_sc_params = pltpu.CompilerParams(use_tc_tiling_on_sc=False)


def _chunk_range(wid):
    c0 = (wid * NCH) // NW
    c1 = ((wid + 1) * NCH) // NW
    return c0, c1 - c0


DLANES = 32

@functools.partial(
    pl.kernel,
    out_type=jax.ShapeDtypeStruct((NC, NPAD, DLANES), jnp.bfloat16),
    mesh=_mesh,
    scratch_types=[
        pltpu.VMEM((CPT, CHUNK), jnp.int32),
        pltpu.VMEM((CHUNK, DLANES), jnp.bfloat16),
        pltpu.VMEM_SHARED((NPAD, DLANES), jnp.bfloat16),
        pltpu.SemaphoreType.DMA,
    ],
    compiler_params=_sc_params,
)
def _deg_kernel(e3, ones_h, zeros_h, out, dst_v, ones_v, acc, ssem):
    c = lax.axis_index("c")
    s = lax.axis_index("s")
    wid = c * NS + s
    c0, tw = _chunk_range(wid)
    pltpu.sync_copy(e3.at[1, pl.ds(c0, CPT)], dst_v)
    pltpu.sync_copy(ones_h, ones_v)
    base = s * RPT
    pltpu.sync_copy(zeros_h.at[pl.ds(base, RPT)], acc.at[pl.ds(base, RPT)])
    plsc.subcore_barrier()

    def fire(j, carry):
        pltpu.async_copy(ones_v, acc.at[dst_v.at[j]], ssem, add=True)
        return carry

    lax.fori_loop(0, tw, fire, 0)

    def drain(j, carry):
        pltpu.make_async_copy(ones_v, acc.at[dst_v.at[0]], ssem).wait()
        return carry

    lax.fori_loop(0, tw, drain, 0)
    plsc.subcore_barrier()
    pltpu.sync_copy(acc.at[pl.ds(base, RPT)], out.at[c, pl.ds(base, RPT)])


def _make_agg(D):
    @functools.partial(
        pl.kernel,
        out_type=jax.ShapeDtypeStruct((NC, NPAD, D), jnp.bfloat16),
        mesh=_mesh,
        scratch_types=[
            pltpu.VMEM((CPT, CHUNK), jnp.int32),
            pltpu.VMEM((CPT, CHUNK), jnp.int32),
            [pltpu.VMEM((CHUNK, D), jnp.bfloat16)] * NRING,
            pltpu.VMEM_SHARED((NPAD, D), jnp.bfloat16),
            [pltpu.SemaphoreType.DMA] * NRING,
            [pltpu.SemaphoreType.DMA] * NRING,
        ],
        compiler_params=_sc_params,
    )
    def agg(e3, h_hbm, zeros_h, out, src_v, dst_v, bufs, acc, gs, ss):
        c = lax.axis_index("c")
        s = lax.axis_index("s")
        wid = c * NS + s
        c0, tw = _chunk_range(wid)
        pltpu.sync_copy(e3.at[0, pl.ds(c0, CPT)], src_v)
        pltpu.sync_copy(e3.at[1, pl.ds(c0, CPT)], dst_v)
        base = s * RPT
        pltpu.sync_copy(zeros_h.at[pl.ds(base, RPT)], acc.at[pl.ds(base, RPT)])
        plsc.subcore_barrier()

        for b in range(NRING):
            pltpu.async_copy(h_hbm.at[src_v.at[b]], bufs[b], gs[b])

        def body(t, carry):
            for b in range(NRING):
                j = NRING * t + b

                @pl.when(j < tw)
                def _():
                    pltpu.make_async_copy(h_hbm.at[src_v.at[j]], bufs[b], gs[b]).wait()
                    pltpu.async_copy(bufs[b], acc.at[dst_v.at[j]], ss[b], add=True)

            for b in range(NRING):
                jn = NRING * t + b + NRING

                @pl.when(jn < tw)
                def _():
                    pltpu.make_async_copy(bufs[b], acc.at[dst_v.at[0]], ss[b]).wait()
                    pltpu.async_copy(h_hbm.at[src_v.at[jn]], bufs[b], gs[b])

            return carry

        lax.fori_loop(0, -(-CPT // NRING), body, 0)
        for b in range(NRING):
            pltpu.make_async_copy(bufs[b], acc.at[dst_v.at[0]], ss[b]).wait()
        plsc.subcore_barrier()
        pltpu.sync_copy(acc.at[pl.ds(base, RPT)], out.at[c, pl.ds(base, RPT)])

    return agg


_agg = _make_agg(D_HID)


BR = 5056
GRID = NPAD // BR


def _dinv_of(degp_ref):
    deg = degp_ref[0, :, 0:1].astype(jnp.float32) + degp_ref[1, :, 0:1].astype(jnp.float32) + 1.0
    return lax.rsqrt(deg)


def _mm1_body(degp_ref, x_ref, w1_ref, h1p_ref, dinv_ref):
    i = pl.program_id(0)
    dinv = _dinv_of(degp_ref)
    h = jnp.dot(x_ref[...], w1_ref[...], preferred_element_type=jnp.float32)
    rows = i * BR + lax.broadcasted_iota(jnp.int32, (BR, 1), 0)
    h1p_ref[...] = jnp.where(rows < N, h * dinv, 0.0).astype(jnp.bfloat16)
    dinv_ref[...] = jnp.broadcast_to(dinv, (BR, 8))


def _mm2_body(dinv_ref, y_ref, h1p_ref, b1_ref, w2_ref, h2p_ref):
    dinv = dinv_ref[:, 0:1]
    ysum = (y_ref[0] + y_ref[1]).astype(jnp.float32) + h1p_ref[...].astype(jnp.float32)
    agg = dinv * ysum + b1_ref[...]
    o = jnp.maximum(agg, 0.0)
    h2 = jnp.dot(o, w2_ref[...], preferred_element_type=jnp.float32) * dinv
    h2p_ref[...] = jnp.pad(h2, ((0, 0), (0, D2 - N_CLASSES))).astype(jnp.bfloat16)


def _out_body(dinv_ref, y_ref, h2p_ref, b2_ref, out_ref):
    dinv = dinv_ref[:, 0:1]
    ysum = (y_ref[0] + y_ref[1]).astype(jnp.float32) + h2p_ref[...].astype(jnp.float32)
    h = (dinv * ysum)[:, :N_CLASSES] + b2_ref[...]
    m = jnp.max(h, axis=1, keepdims=True)
    lse = jnp.log(jnp.sum(jnp.exp(h - m), axis=1, keepdims=True)) + m
    out_ref[...] = h - lse


def _deg_spec():
    return pl.BlockSpec((NC, BR, DLANES), lambda i: (0, i, 0))


_mm1 = pl.pallas_call(
    _mm1_body,
    grid=(GRID,),
    in_specs=[
        _deg_spec(),
        pl.BlockSpec((BR, D_IN), lambda i: (i, 0)),
        pl.BlockSpec((D_IN, D_HID), lambda i: (0, 0)),
    ],
    out_specs=[
        pl.BlockSpec((BR, D_HID), lambda i: (i, 0)),
        pl.BlockSpec((BR, 8), lambda i: (i, 0)),
    ],
    out_shape=[
        jax.ShapeDtypeStruct((NPAD, D_HID), jnp.bfloat16),
        jax.ShapeDtypeStruct((NPAD, 8), jnp.float32),
    ],
)

_mm2 = pl.pallas_call(
    _mm2_body,
    grid=(GRID,),
    in_specs=[
        pl.BlockSpec((BR, 8), lambda i: (i, 0)),
        pl.BlockSpec((NC, BR, D_HID), lambda i: (0, i, 0)),
        pl.BlockSpec((BR, D_HID), lambda i: (i, 0)),
        pl.BlockSpec((1, D_HID), lambda i: (0, 0)),
        pl.BlockSpec((D_HID, N_CLASSES), lambda i: (0, 0)),
    ],
    out_specs=pl.BlockSpec((BR, D2), lambda i: (i, 0)),
    out_shape=jax.ShapeDtypeStruct((NPAD, D2), jnp.bfloat16),
)

_outk = pl.pallas_call(
    _out_body,
    grid=(GRID,),
    in_specs=[
        pl.BlockSpec((BR, 8), lambda i: (i, 0)),
        pl.BlockSpec((NC, BR, D2), lambda i: (0, i, 0)),
        pl.BlockSpec((BR, D2), lambda i: (i, 0)),
        pl.BlockSpec((1, N_CLASSES), lambda i: (0, 0)),
    ],
    out_specs=pl.BlockSpec((BR, N_CLASSES), lambda i: (i, 0)),
    out_shape=jax.ShapeDtypeStruct((N, N_CLASSES), jnp.float32),
)


@jax.jit
def kernel(x, edge_index, W1, b1, W2, b2):
    e3 = edge_index.astype(jnp.int32).reshape(2, NCH, CHUNK)

    ones_h = jnp.ones((CHUNK, DLANES), jnp.bfloat16)
    degp = _deg_kernel(e3, ones_h, jnp.zeros((NPAD, DLANES), jnp.bfloat16))

    h1p, dinv = _mm1(degp, x, W1)
    zeros_bf = jnp.zeros((NPAD, D_HID), jnp.bfloat16)
    y1 = _agg(e3, h1p, zeros_bf)

    h2p = _mm2(dinv, y1, h1p, b1.reshape(1, D_HID), W2)
    y2 = _agg(e3, h2p, zeros_bf)

    return _outk(dinv, y2, h2p, b2.reshape(1, N_CLASSES))
